# baseline (speedup 1.0000x reference)
import functools

import jax
import jax.numpy as jnp
from jax import lax
from jax.experimental import pallas as pl
from jax.experimental.pallas import tpu as pltpu
from jax.experimental.pallas import tpu_sc as plsc

N_USER_C = 25000
N_ITEM_C = 25000
N_NODES = N_USER_C + N_ITEM_C
D = 64
QSIZE = 12800
TRASH = QSIZE
ACC_ROWS = QSIZE + 8
CHUNK = 128
N_SC = 2
N_TILE = 16
N_WORKERS = N_SC * N_TILE

OUT_CHUNK = 200

_SC_PARAMS = pltpu.CompilerParams(needs_layout_passes=False)


def _pad_edges(n_edges):
    ept = -(-n_edges // N_TILE)
    ept = -(-ept // CHUNK) * CHUNK
    return ept


ROWS_PER_TILE = 1568
N_PAD_NODES = N_WORKERS * ROWS_PER_TILE
ACC2 = ROWS_PER_TILE + 16
CH = 256


def _spmm_body(g_hbm, dst_hbm, bounds_hbm, zro_hbm, le_hbm,
               boundsbuf, dstbuf, gbuf, acc, sem):
    c = lax.axis_index("c")
    s = lax.axis_index("s")
    wid = s * N_SC + c
    base = wid * ROWS_PER_TILE
    iota = lax.iota(jnp.int32, 16)

    pltpu.sync_copy(zro_hbm, acc)
    pltpu.sync_copy(bounds_hbm, boundsbuf)
    wsplat = jnp.full((16,), wid, jnp.int32)
    lo = plsc.load_gather(boundsbuf, [wsplat])[0]
    hi = plsc.load_gather(boundsbuf, [wsplat + 1])[0]
    lo16 = (lo // 16) * 16
    nch = (hi - lo16 + CH - 1) // CH

    @pl.loop(0, nch)
    def _chunk(j):
        off = pl.multiple_of(lo16 + j * CH, 16)
        pltpu.sync_copy(g_hbm.at[pl.ds(pl.multiple_of(off // 2, 8), CH // 2)],
                        gbuf)
        pltpu.sync_copy(dst_hbm.at[pl.ds(off, CH)], dstbuf)

        for k in range(CH // 16):
            d = dstbuf[pl.ds(k * 16, 16)] - base
            ok = (d >= 0) & (d < ROWS_PER_TILE)
            dstbuf[pl.ds(k * 16, 16)] = jnp.where(ok, d, ROWS_PER_TILE)

        @pl.loop(0, CH, unroll=4)
        def _add(e):
            esplat = jnp.full((16,), e, jnp.int32)
            dsplat = plsc.load_gather(dstbuf, [esplat])
            drow = dsplat >> 1
            dlane = (dsplat & 1) * D
            erow = esplat >> 1
            elane = (esplat & 1) * D
            for q in range(D // 16):
                lane = (q * 16) + iota
                gval = plsc.load_gather(gbuf, [erow, elane + lane])
                plsc.addupdate_scatter(acc, [drow, dlane + lane], gval)

    pltpu.sync_copy(acc.at[pl.ds(0, ROWS_PER_TILE // 2)],
                    le_hbm.at[pl.ds(pl.multiple_of(base // 2, 8),
                                    ROWS_PER_TILE // 2)])


def _spmm(g, dst_s, bounds, zro):
    mesh = plsc.VectorSubcoreMesh(core_axis_name="c", subcore_axis_name="s", num_cores=N_SC, num_subcores=N_TILE)
    f = pl.kernel(
        _spmm_body,
        out_type=jax.ShapeDtypeStruct((N_PAD_NODES // 2, 2 * D), jnp.float32),
        mesh=mesh,
        compiler_params=_SC_PARAMS,
        scratch_types=[
            pltpu.VMEM((48,), jnp.int32),
            pltpu.VMEM((CH,), jnp.int32),
            pltpu.VMEM((CH // 2, 2 * D), jnp.float32),
            pltpu.VMEM((ACC2 // 2, 2 * D), jnp.float32),
            pltpu.SemaphoreType.DMA,
        ],
    )
    return f(g, dst_s, bounds, zro)


def _dense_body(e_ref, le_ref, w1t_ref, b1_ref, w2t_ref, b2_ref,
                enext_ref, g_ref):
    e = e_ref[...]
    le = le_ref[...]
    x = jnp.dot(le + e, w1t_ref[...], preferred_element_type=jnp.float32)
    y = jnp.dot(le * e, w2t_ref[...], preferred_element_type=jnp.float32)
    snew = x + y + 2.0 * b1_ref[...] + b2_ref[...]
    out = jnp.where(snew >= 0, snew, 0.2 * snew)
    enext_ref[...] = out
    nrm = jnp.sqrt(jnp.sum(out * out, axis=1, keepdims=True))
    g_ref[...] = out / jnp.maximum(nrm, 1e-12)


def _dense(e, le, w1t, b1, w2t, b2):
    blk = 2000
    grid = (N_NODES // blk,)
    return pl.pallas_call(
        _dense_body,
        grid=grid,
        in_specs=[
            pl.BlockSpec((blk, D), lambda i: (i, 0)),
            pl.BlockSpec((blk, D), lambda i: (i, 0)),
            pl.BlockSpec((D, D), lambda i: (0, 0)),
            pl.BlockSpec((1, D), lambda i: (0, 0)),
            pl.BlockSpec((D, D), lambda i: (0, 0)),
            pl.BlockSpec((1, D), lambda i: (0, 0)),
        ],
        out_specs=[
            pl.BlockSpec((blk, D), lambda i: (i, 0)),
            pl.BlockSpec((blk, D), lambda i: (i, 0)),
        ],
        out_shape=[
            jax.ShapeDtypeStruct((N_NODES, D), jnp.float32),
            jax.ShapeDtypeStruct((N_NODES, D), jnp.float32),
        ],
    )(e, le, w1t, b1, w2t, b2)


def _leaky(x, a):
    return jnp.where(x >= 0, x, a * x)


def _mlp_body(feats_ref, l1wt_ref, l1b_ref, l2wt_ref, l2b_ref, urow_ref,
              out_ref):
    f = feats_ref[...]
    h = jnp.dot(f, l1wt_ref[...],
                preferred_element_type=jnp.float32) + l1b_ref[...]
    h = _leaky(h, 0.01)
    u = jnp.dot(h, l2wt_ref[...],
                preferred_element_type=jnp.float32) + l2b_ref[...]
    u = _leaky(u, 0.01)
    out_ref[...] = urow_ref[...] * 0.5 + u * 0.5


def _blended_row(feats, lin1_w, lin1_b, lin2_w, lin2_b, urow):
    return pl.pallas_call(
        _mlp_body,
        out_shape=jax.ShapeDtypeStruct((1, D), jnp.float32),
    )(feats.reshape(1, -1), lin1_w.T, lin1_b.reshape(1, -1),
      lin2_w.T, lin2_b.reshape(1, -1), urow.reshape(1, D))


def _gather_body(g0, g1, g2, g3, gidxp, gidxoff, o0, o1, o2,
                 idxbuf, offbuf, rowsbuf, outbuf, sem):
    c = lax.axis_index("c")
    s = lax.axis_index("s")
    wid = s * N_SC + c
    rpw = 1024 // N_WORKERS
    base = wid * rpw
    iota = lax.iota(jnp.int32, 16)
    tables = (g0, g1, g2, g3)
    outs = (o0, o1, o2)
    for x in range(3):
        pltpu.sync_copy(gidxp.at[0, pl.ds(x * 1024 + base, rpw)], idxbuf.at[0])
        pltpu.sync_copy(gidxoff.at[pl.ds(x * 1024 + base, rpw)], offbuf)
        for t in range(4):
            pltpu.async_copy(tables[t].at[idxbuf.at[0]], rowsbuf, sem).wait()

            @pl.loop(0, rpw)
            def _extract(r):
                rsplat = jnp.full((16,), r, jnp.int32)
                hoff = plsc.load_gather(offbuf, [rsplat])
                for q in range(D // 16):
                    v = plsc.load_gather(rowsbuf,
                                         [rsplat, hoff + (q * 16) + iota])
                    outbuf[r, pl.ds(t * D + q * 16, 16)] = v

        pltpu.sync_copy(outbuf, outs[x].at[pl.ds(base, rpw)])


def _gather_out(g0p, g1p, g2p, g3p, gidxp, gidxoff):
    rpw = 1024 // N_WORKERS
    mesh = plsc.VectorSubcoreMesh(core_axis_name="c", subcore_axis_name="s", num_cores=N_SC, num_subcores=N_TILE)
    f = pl.kernel(
        _gather_body,
        out_type=[jax.ShapeDtypeStruct((1024, 4 * D), jnp.float32)] * 3,
        mesh=mesh,
        compiler_params=_SC_PARAMS,
        scratch_types=[
            pltpu.VMEM((1, rpw), jnp.int32),
            pltpu.VMEM((rpw,), jnp.int32),
            pltpu.VMEM((rpw, 2 * D), jnp.float32),
            pltpu.VMEM((rpw, 4 * D), jnp.float32),
            pltpu.SemaphoreType.DMA,
        ],
    )
    return f(g0p, g1p, g2p, g3p, gidxp, gidxoff)


def kernel(year, u_id, age, date, sex, pos_item, neg_item, node_flag,
           lap_indices, lap_values, user_table, item_table, age_table,
           date_table, sex_table, lin1_w, lin1_b, lin2_w, lin2_b,
           w1_w, w1_b, w2_w, w2_b):
    n_user = user_table.shape[0]
    n_edges = lap_indices.shape[1]

    feats = jnp.concatenate([
        lax.dynamic_slice_in_dim(age_table, age[0], 1, 0).reshape(-1),
        lax.dynamic_slice_in_dim(date_table, date[0], 1, 0).reshape(-1),
        lax.dynamic_slice_in_dim(sex_table, sex[0], 1, 0).reshape(-1),
    ], axis=0)
    uid0 = u_id[0]
    urow0 = lax.dynamic_slice_in_dim(user_table, uid0, 1, 0)
    brow = _blended_row(feats, lin1_w, lin1_b, lin2_w, lin2_b, urow0)
    user_table = lax.dynamic_update_slice(user_table, brow,
                                          (uid0, jnp.int32(0)))
    e0 = jnp.concatenate([user_table, item_table], axis=0)

    ept = _pad_edges(n_edges)
    pad = N_TILE * ept - n_edges
    row_arr = jnp.pad(lap_indices[0], (0, pad), constant_values=N_PAD_NODES)
    col = jnp.pad(lap_indices[1], (0, pad))
    val = jnp.pad(lap_values, (0, pad))
    perm = jnp.arange(row_arr.shape[0])
    dst_s = row_arr[perm].astype(jnp.int32)
    col_s = col[perm]
    val_s = val[perm]
    dst_s = jnp.pad(dst_s, (0, CH), constant_values=N_PAD_NODES)
    starts = jnp.arange(N_WORKERS + 1, dtype=jnp.int32) * ROWS_PER_TILE
    bounds = jnp.searchsorted(dst_s[:-CH], starts).astype(jnp.int32)
    bounds = jnp.pad(bounds, (0, 48 - N_WORKERS - 1))
    zro = jnp.zeros((ACC2 // 2, 2 * D), jnp.float32)

    e = e0
    gs = [e0]
    for i in range(3):
        g_rows = val_s[:, None] * e[col_s]
        g_rows = jnp.pad(g_rows, ((0, CH), (0, 0))).reshape(-1, 2 * D)
        le = _spmm(g_rows, dst_s, bounds, zro).reshape(-1, D)[:N_NODES]
        e, g = _dense(e, le, w1_w[i].T, w1_b[i].reshape(1, D),
                      w2_w[i].T, w2_b[i].reshape(1, D))
        gs.append(g)

    gidx = jnp.concatenate([
        u_id.astype(jnp.int32),
        (n_user + pos_item).astype(jnp.int32),
        (n_user + neg_item).astype(jnp.int32),
    ], axis=0)
    gidxp = (gidx >> 1).reshape(1, -1)
    gidxoff = (gidx & 1) * D
    gps = [g.reshape(N_NODES // 2, 2 * D) for g in gs]
    o0, o1, o2 = _gather_out(gps[0], gps[1], gps[2], gps[3], gidxp, gidxoff)
    return (o0, o1, o2)


# speedup vs baseline: 3.3130x; 3.3130x over previous
import functools

import jax
import jax.numpy as jnp
from jax import lax
from jax.experimental import pallas as pl
from jax.experimental.pallas import tpu as pltpu
from jax.experimental.pallas import tpu_sc as plsc

N_USER_C = 25000
N_ITEM_C = 25000
N_NODES = N_USER_C + N_ITEM_C
D = 64
QSIZE = 12800
TRASH = QSIZE
ACC_ROWS = QSIZE + 8
CHUNK = 128
N_SC = 2
N_TILE = 16
N_WORKERS = N_SC * N_TILE

OUT_CHUNK = 200

_SC_PARAMS = pltpu.CompilerParams(needs_layout_passes=False)


def _pad_edges(n_edges):
    ept = -(-n_edges // N_TILE)
    ept = -(-ept // CHUNK) * CHUNK
    return ept


ROWS_PER_TILE = 1568
N_PAD_NODES = N_WORKERS * ROWS_PER_TILE
ACC2 = ROWS_PER_TILE + 16
CH = 256


def _spmm_body(g_hbm, dst_hbm, bounds_hbm, zro_hbm, le_hbm,
               boundsbuf, dstbuf, gbuf, acc, sem):
    c = lax.axis_index("c")
    s = lax.axis_index("s")
    wid = s * N_SC + c
    base = wid * ROWS_PER_TILE
    iota = lax.iota(jnp.int32, 16)

    pltpu.sync_copy(zro_hbm, acc)
    pltpu.sync_copy(bounds_hbm, boundsbuf)
    wsplat = jnp.full((16,), wid, jnp.int32)
    lo = plsc.load_gather(boundsbuf, [wsplat])[0]
    hi = plsc.load_gather(boundsbuf, [wsplat + 1])[0]
    lo16 = (lo // 16) * 16
    nch = (hi - lo16 + CH - 1) // CH

    @pl.loop(0, nch)
    def _chunk(j):
        off = pl.multiple_of(lo16 + j * CH, 16)
        pltpu.sync_copy(g_hbm.at[pl.ds(pl.multiple_of(off // 2, 8), CH // 2)],
                        gbuf)
        pltpu.sync_copy(dst_hbm.at[pl.ds(off, CH)], dstbuf)

        for k in range(CH // 16):
            d = dstbuf[pl.ds(k * 16, 16)] - base
            ok = (d >= 0) & (d < ROWS_PER_TILE)
            dstbuf[pl.ds(k * 16, 16)] = jnp.where(ok, d, ROWS_PER_TILE)

        @pl.loop(0, CH, unroll=8)
        def _add(e):
            esplat = jnp.full((16,), e, jnp.int32)
            dsplat = plsc.load_gather(dstbuf, [esplat])
            drow = dsplat >> 1
            dlane = (dsplat & 1) * D
            erow = esplat >> 1
            elane = (esplat & 1) * D
            for q in range(D // 16):
                lane = (q * 16) + iota
                gval = plsc.load_gather(gbuf, [erow, elane + lane])
                plsc.addupdate_scatter(acc, [drow, dlane + lane], gval)

    pltpu.sync_copy(acc.at[pl.ds(0, ROWS_PER_TILE // 2)],
                    le_hbm.at[pl.ds(pl.multiple_of(base // 2, 8),
                                    ROWS_PER_TILE // 2)])


def _spmm(g, dst_s, bounds, zro):
    mesh = plsc.VectorSubcoreMesh(core_axis_name="c", subcore_axis_name="s", num_cores=N_SC, num_subcores=N_TILE)
    f = pl.kernel(
        _spmm_body,
        out_type=jax.ShapeDtypeStruct((N_PAD_NODES // 2, 2 * D), jnp.float32),
        mesh=mesh,
        compiler_params=_SC_PARAMS,
        scratch_types=[
            pltpu.VMEM((48,), jnp.int32),
            pltpu.VMEM((CH,), jnp.int32),
            pltpu.VMEM((CH // 2, 2 * D), jnp.float32),
            pltpu.VMEM((ACC2 // 2, 2 * D), jnp.float32),
            pltpu.SemaphoreType.DMA,
        ],
    )
    return f(g, dst_s, bounds, zro)


def _dense_body(e_ref, le_ref, w1t_ref, b1_ref, w2t_ref, b2_ref,
                enext_ref, g_ref):
    e = e_ref[...]
    le = le_ref[...]
    x = jnp.dot(le + e, w1t_ref[...], preferred_element_type=jnp.float32)
    y = jnp.dot(le * e, w2t_ref[...], preferred_element_type=jnp.float32)
    snew = x + y + 2.0 * b1_ref[...] + b2_ref[...]
    out = jnp.where(snew >= 0, snew, 0.2 * snew)
    enext_ref[...] = out
    nrm = jnp.sqrt(jnp.sum(out * out, axis=1, keepdims=True))
    g_ref[...] = out / jnp.maximum(nrm, 1e-12)


def _dense(e, le, w1t, b1, w2t, b2):
    blk = 2000
    grid = (N_NODES // blk,)
    return pl.pallas_call(
        _dense_body,
        grid=grid,
        in_specs=[
            pl.BlockSpec((blk, D), lambda i: (i, 0)),
            pl.BlockSpec((blk, D), lambda i: (i, 0)),
            pl.BlockSpec((D, D), lambda i: (0, 0)),
            pl.BlockSpec((1, D), lambda i: (0, 0)),
            pl.BlockSpec((D, D), lambda i: (0, 0)),
            pl.BlockSpec((1, D), lambda i: (0, 0)),
        ],
        out_specs=[
            pl.BlockSpec((blk, D), lambda i: (i, 0)),
            pl.BlockSpec((blk, D), lambda i: (i, 0)),
        ],
        out_shape=[
            jax.ShapeDtypeStruct((N_NODES, D), jnp.float32),
            jax.ShapeDtypeStruct((N_NODES, D), jnp.float32),
        ],
    )(e, le, w1t, b1, w2t, b2)


def _leaky(x, a):
    return jnp.where(x >= 0, x, a * x)


def _mlp_body(feats_ref, l1wt_ref, l1b_ref, l2wt_ref, l2b_ref, urow_ref,
              out_ref):
    f = feats_ref[...]
    h = jnp.dot(f, l1wt_ref[...],
                preferred_element_type=jnp.float32) + l1b_ref[...]
    h = _leaky(h, 0.01)
    u = jnp.dot(h, l2wt_ref[...],
                preferred_element_type=jnp.float32) + l2b_ref[...]
    u = _leaky(u, 0.01)
    out_ref[...] = urow_ref[...] * 0.5 + u * 0.5


def _blended_row(feats, lin1_w, lin1_b, lin2_w, lin2_b, urow):
    return pl.pallas_call(
        _mlp_body,
        out_shape=jax.ShapeDtypeStruct((1, D), jnp.float32),
    )(feats.reshape(1, -1), lin1_w.T, lin1_b.reshape(1, -1),
      lin2_w.T, lin2_b.reshape(1, -1), urow.reshape(1, D))


def _gather_body(g0, g1, g2, g3, gidxp, gidxoff, o0, o1, o2,
                 idxbuf, offbuf, rowsbuf, outbuf, sem):
    c = lax.axis_index("c")
    s = lax.axis_index("s")
    wid = s * N_SC + c
    rpw = 1024 // N_WORKERS
    base = wid * rpw
    iota = lax.iota(jnp.int32, 16)
    tables = (g0, g1, g2, g3)
    outs = (o0, o1, o2)
    for x in range(3):
        pltpu.sync_copy(gidxp.at[0, pl.ds(x * 1024 + base, rpw)], idxbuf.at[0])
        pltpu.sync_copy(gidxoff.at[pl.ds(x * 1024 + base, rpw)], offbuf)
        for t in range(4):
            pltpu.async_copy(tables[t].at[idxbuf.at[0]], rowsbuf, sem).wait()

            @pl.loop(0, rpw)
            def _extract(r):
                rsplat = jnp.full((16,), r, jnp.int32)
                hoff = plsc.load_gather(offbuf, [rsplat])
                for q in range(D // 16):
                    v = plsc.load_gather(rowsbuf,
                                         [rsplat, hoff + (q * 16) + iota])
                    outbuf[r, pl.ds(t * D + q * 16, 16)] = v

        pltpu.sync_copy(outbuf, outs[x].at[pl.ds(base, rpw)])


def _gather_out(g0p, g1p, g2p, g3p, gidxp, gidxoff):
    rpw = 1024 // N_WORKERS
    mesh = plsc.VectorSubcoreMesh(core_axis_name="c", subcore_axis_name="s", num_cores=N_SC, num_subcores=N_TILE)
    f = pl.kernel(
        _gather_body,
        out_type=[jax.ShapeDtypeStruct((1024, 4 * D), jnp.float32)] * 3,
        mesh=mesh,
        compiler_params=_SC_PARAMS,
        scratch_types=[
            pltpu.VMEM((1, rpw), jnp.int32),
            pltpu.VMEM((rpw,), jnp.int32),
            pltpu.VMEM((rpw, 2 * D), jnp.float32),
            pltpu.VMEM((rpw, 4 * D), jnp.float32),
            pltpu.SemaphoreType.DMA,
        ],
    )
    return f(g0p, g1p, g2p, g3p, gidxp, gidxoff)


def kernel(year, u_id, age, date, sex, pos_item, neg_item, node_flag,
           lap_indices, lap_values, user_table, item_table, age_table,
           date_table, sex_table, lin1_w, lin1_b, lin2_w, lin2_b,
           w1_w, w1_b, w2_w, w2_b):
    n_user = user_table.shape[0]
    n_edges = lap_indices.shape[1]

    feats = jnp.concatenate([
        lax.dynamic_slice_in_dim(age_table, age[0], 1, 0).reshape(-1),
        lax.dynamic_slice_in_dim(date_table, date[0], 1, 0).reshape(-1),
        lax.dynamic_slice_in_dim(sex_table, sex[0], 1, 0).reshape(-1),
    ], axis=0)
    uid0 = u_id[0]
    urow0 = lax.dynamic_slice_in_dim(user_table, uid0, 1, 0)
    brow = _blended_row(feats, lin1_w, lin1_b, lin2_w, lin2_b, urow0)
    user_table = lax.dynamic_update_slice(user_table, brow,
                                          (uid0, jnp.int32(0)))
    e0 = jnp.concatenate([user_table, item_table], axis=0)

    ept = _pad_edges(n_edges)
    pad = N_TILE * ept - n_edges
    row_arr = jnp.pad(lap_indices[0], (0, pad), constant_values=N_PAD_NODES)
    col = jnp.pad(lap_indices[1], (0, pad))
    val = jnp.pad(lap_values, (0, pad))
    dst_s, col_s, val_s = lax.sort([row_arr, col, val], num_keys=1)
    dst_s = dst_s.astype(jnp.int32)
    dst_s = jnp.pad(dst_s, (0, CH), constant_values=N_PAD_NODES)
    starts = jnp.arange(N_WORKERS + 1, dtype=jnp.int32) * ROWS_PER_TILE
    bounds = jnp.searchsorted(dst_s[:-CH], starts).astype(jnp.int32)
    bounds = jnp.pad(bounds, (0, 48 - N_WORKERS - 1))
    zro = jnp.zeros((ACC2 // 2, 2 * D), jnp.float32)

    e = e0
    gs = [e0]
    for i in range(3):
        g_rows = val_s[:, None] * e[col_s]
        g_rows = jnp.pad(g_rows, ((0, CH), (0, 0))).reshape(-1, 2 * D)
        le = _spmm(g_rows, dst_s, bounds, zro).reshape(-1, D)[:N_NODES]
        e, g = _dense(e, le, w1_w[i].T, w1_b[i].reshape(1, D),
                      w2_w[i].T, w2_b[i].reshape(1, D))
        gs.append(g)

    gidx = jnp.concatenate([
        u_id.astype(jnp.int32),
        (n_user + pos_item).astype(jnp.int32),
        (n_user + neg_item).astype(jnp.int32),
    ], axis=0)
    gidxp = (gidx >> 1).reshape(1, -1)
    gidxoff = (gidx & 1) * D
    gps = [g.reshape(N_NODES // 2, 2 * D) for g in gs]
    o0, o1, o2 = _gather_out(gps[0], gps[1], gps[2], gps[3], gidxp, gidxoff)
    return (o0, o1, o2)


# speedup vs baseline: 3.3718x; 1.0178x over previous
import jax
import jax.numpy as jnp
from jax import lax
from jax.experimental import pallas as pl
from jax.experimental.pallas import tpu as pltpu
from jax.experimental.pallas import tpu_sc as plsc

N_USER_C = 25000
N_ITEM_C = 25000
N_NODES = N_USER_C + N_ITEM_C
D = 64
N_SC = 2
N_TILE = 16
N_WORKERS = N_SC * N_TILE
CHUNK = 128

_SC_PARAMS = pltpu.CompilerParams(needs_layout_passes=False)


def _pad_edges(n_edges):
    ept = -(-n_edges // N_TILE)
    ept = -(-ept // CHUNK) * CHUNK
    return ept


ROWS_PER_TILE = 1568
N_PAD_NODES = N_WORKERS * ROWS_PER_TILE
ACC2 = ROWS_PER_TILE + 16
CH = 256


def _spmm_body(g_hbm, dst_hbm, bounds_hbm, zro_hbm, le_hbm,
               boundsbuf, dstbuf, gbuf, acc, sem):
    c = lax.axis_index("c")
    s = lax.axis_index("s")
    wid = s * N_SC + c
    base = wid * ROWS_PER_TILE
    iota = lax.iota(jnp.int32, 16)

    pltpu.sync_copy(zro_hbm, acc)
    pltpu.sync_copy(bounds_hbm, boundsbuf)
    wsplat = jnp.full((16,), wid, jnp.int32)
    lo = plsc.load_gather(boundsbuf, [wsplat])[0]
    hi = plsc.load_gather(boundsbuf, [wsplat + 1])[0]
    lo16 = (lo // 16) * 16
    nch = (hi - lo16 + CH - 1) // CH

    @pl.loop(0, nch)
    def _chunk(j):
        off = pl.multiple_of(lo16 + j * CH, 16)
        pltpu.sync_copy(g_hbm.at[pl.ds(pl.multiple_of(off // 2, 8), CH // 2)],
                        gbuf)
        pltpu.sync_copy(dst_hbm.at[pl.ds(off, CH)], dstbuf)

        for k in range(CH // 16):
            d = dstbuf[pl.ds(k * 16, 16)] - base
            ok = (d >= 0) & (d < ROWS_PER_TILE)
            dstbuf[pl.ds(k * 16, 16)] = jnp.where(ok, d, ROWS_PER_TILE)

        @pl.loop(0, CH, unroll=4)
        def _add(e):
            esplat = jnp.full((16,), e, jnp.int32)
            dsplat = plsc.load_gather(dstbuf, [esplat])
            drow = dsplat >> 1
            dlane = (dsplat & 1) * D
            erow = esplat >> 1
            elane = (esplat & 1) * D
            for q in range(D // 16):
                lane = (q * 16) + iota
                gval = plsc.load_gather(gbuf, [erow, elane + lane])
                plsc.addupdate_scatter(acc, [drow, dlane + lane], gval)

    pltpu.sync_copy(acc.at[pl.ds(0, ROWS_PER_TILE // 2)],
                    le_hbm.at[pl.ds(pl.multiple_of(base // 2, 8),
                                    ROWS_PER_TILE // 2)])


def _spmm(g, dst_s, bounds, zro):
    mesh = plsc.VectorSubcoreMesh(core_axis_name="c", subcore_axis_name="s", num_cores=N_SC, num_subcores=N_TILE)
    f = pl.kernel(
        _spmm_body,
        out_type=jax.ShapeDtypeStruct((N_PAD_NODES // 2, 2 * D), jnp.float32),
        mesh=mesh,
        compiler_params=_SC_PARAMS,
        scratch_types=[
            pltpu.VMEM((48,), jnp.int32),
            pltpu.VMEM((CH,), jnp.int32),
            pltpu.VMEM((CH // 2, 2 * D), jnp.float32),
            pltpu.VMEM((ACC2 // 2, 2 * D), jnp.float32),
            pltpu.SemaphoreType.DMA,
        ],
    )
    return f(g, dst_s, bounds, zro)


def _dense_body(e_ref, le_ref, w1t_ref, b1_ref, w2t_ref, b2_ref,
                enext_ref, g_ref):
    e = e_ref[...]
    le = le_ref[...]
    x = jnp.dot(le + e, w1t_ref[...], preferred_element_type=jnp.float32)
    y = jnp.dot(le * e, w2t_ref[...], preferred_element_type=jnp.float32)
    snew = x + y + 2.0 * b1_ref[...] + b2_ref[...]
    out = jnp.where(snew >= 0, snew, 0.2 * snew)
    enext_ref[...] = out
    nrm = jnp.sqrt(jnp.sum(out * out, axis=1, keepdims=True))
    g_ref[...] = out / jnp.maximum(nrm, 1e-12)


def _dense(e, le, w1t, b1, w2t, b2):
    blk = 2000
    grid = (N_NODES // blk,)
    return pl.pallas_call(
        _dense_body,
        grid=grid,
        in_specs=[
            pl.BlockSpec((blk, D), lambda i: (i, 0)),
            pl.BlockSpec((blk, D), lambda i: (i, 0)),
            pl.BlockSpec((D, D), lambda i: (0, 0)),
            pl.BlockSpec((1, D), lambda i: (0, 0)),
            pl.BlockSpec((D, D), lambda i: (0, 0)),
            pl.BlockSpec((1, D), lambda i: (0, 0)),
        ],
        out_specs=[
            pl.BlockSpec((blk, D), lambda i: (i, 0)),
            pl.BlockSpec((blk, D), lambda i: (i, 0)),
        ],
        out_shape=[
            jax.ShapeDtypeStruct((N_NODES, D), jnp.float32),
            jax.ShapeDtypeStruct((N_NODES, D), jnp.float32),
        ],
    )(e, le, w1t, b1, w2t, b2)


def _leaky(x, a):
    return jnp.where(x >= 0, x, a * x)


def _mlp_body(feats_ref, l1wt_ref, l1b_ref, l2wt_ref, l2b_ref, urow_ref,
              out_ref):
    f = feats_ref[...]
    h = jnp.dot(f, l1wt_ref[...],
                preferred_element_type=jnp.float32) + l1b_ref[...]
    h = _leaky(h, 0.01)
    u = jnp.dot(h, l2wt_ref[...],
                preferred_element_type=jnp.float32) + l2b_ref[...]
    u = _leaky(u, 0.01)
    out_ref[...] = urow_ref[...] * 0.5 + u * 0.5


def _blended_row(feats, lin1_w, lin1_b, lin2_w, lin2_b, urow):
    return pl.pallas_call(
        _mlp_body,
        out_shape=jax.ShapeDtypeStruct((1, D), jnp.float32),
    )(feats.reshape(1, -1), lin1_w.T, lin1_b.reshape(1, -1),
      lin2_w.T, lin2_b.reshape(1, -1), urow.reshape(1, D))


def _gather_body(g0, g1, g2, g3, gidxp, gidxoff, o0, o1, o2,
                 idxbuf, offbuf, rowsbuf, outbuf, sem):
    c = lax.axis_index("c")
    s = lax.axis_index("s")
    wid = s * N_SC + c
    rpw = 1024 // N_WORKERS
    base = wid * rpw
    iota = lax.iota(jnp.int32, 16)
    tables = (g0, g1, g2, g3)
    outs = (o0, o1, o2)
    for x in range(3):
        pltpu.sync_copy(gidxp.at[0, pl.ds(x * 1024 + base, rpw)], idxbuf.at[0])
        pltpu.sync_copy(gidxoff.at[pl.ds(x * 1024 + base, rpw)], offbuf)
        for t in range(4):
            pltpu.async_copy(tables[t].at[idxbuf.at[0]], rowsbuf, sem).wait()

            @pl.loop(0, rpw)
            def _extract(r):
                rsplat = jnp.full((16,), r, jnp.int32)
                hoff = plsc.load_gather(offbuf, [rsplat])
                for q in range(D // 16):
                    v = plsc.load_gather(rowsbuf,
                                         [rsplat, hoff + (q * 16) + iota])
                    outbuf[r, pl.ds(t * D + q * 16, 16)] = v

        pltpu.sync_copy(outbuf, outs[x].at[pl.ds(base, rpw)])


def _gather_out(g0p, g1p, g2p, g3p, gidxp, gidxoff):
    rpw = 1024 // N_WORKERS
    mesh = plsc.VectorSubcoreMesh(core_axis_name="c", subcore_axis_name="s", num_cores=N_SC, num_subcores=N_TILE)
    f = pl.kernel(
        _gather_body,
        out_type=[jax.ShapeDtypeStruct((1024, 4 * D), jnp.float32)] * 3,
        mesh=mesh,
        compiler_params=_SC_PARAMS,
        scratch_types=[
            pltpu.VMEM((1, rpw), jnp.int32),
            pltpu.VMEM((rpw,), jnp.int32),
            pltpu.VMEM((rpw, 2 * D), jnp.float32),
            pltpu.VMEM((rpw, 4 * D), jnp.float32),
            pltpu.SemaphoreType.DMA,
        ],
    )
    return f(g0p, g1p, g2p, g3p, gidxp, gidxoff)


def kernel(year, u_id, age, date, sex, pos_item, neg_item, node_flag,
           lap_indices, lap_values, user_table, item_table, age_table,
           date_table, sex_table, lin1_w, lin1_b, lin2_w, lin2_b,
           w1_w, w1_b, w2_w, w2_b):
    n_user = user_table.shape[0]
    n_edges = lap_indices.shape[1]

    feats = jnp.concatenate([
        lax.dynamic_slice_in_dim(age_table, age[0], 1, 0).reshape(-1),
        lax.dynamic_slice_in_dim(date_table, date[0], 1, 0).reshape(-1),
        lax.dynamic_slice_in_dim(sex_table, sex[0], 1, 0).reshape(-1),
    ], axis=0)
    uid0 = u_id[0]
    urow0 = lax.dynamic_slice_in_dim(user_table, uid0, 1, 0)
    brow = _blended_row(feats, lin1_w, lin1_b, lin2_w, lin2_b, urow0)
    user_table = lax.dynamic_update_slice(user_table, brow,
                                          (uid0, jnp.int32(0)))
    e0 = jnp.concatenate([user_table, item_table], axis=0)

    ept = _pad_edges(n_edges)
    pad = N_TILE * ept - n_edges
    row_arr = jnp.pad(lap_indices[0], (0, pad), constant_values=N_PAD_NODES)
    col = jnp.pad(lap_indices[1], (0, pad))
    val = jnp.pad(lap_values, (0, pad))
    perm = jnp.argsort(row_arr)
    dst_s = row_arr[perm].astype(jnp.int32)
    col_s = col[perm]
    val_s = val[perm]
    dst_s = jnp.pad(dst_s, (0, CH), constant_values=N_PAD_NODES)
    starts = jnp.arange(N_WORKERS + 1, dtype=jnp.int32) * ROWS_PER_TILE
    bounds = jnp.searchsorted(dst_s[:-CH], starts).astype(jnp.int32)
    bounds = jnp.pad(bounds, (0, 48 - N_WORKERS - 1))
    zro = jnp.zeros((ACC2 // 2, 2 * D), jnp.float32)

    e = e0
    gs = [e0]
    for i in range(3):
        g_rows = val_s[:, None] * e[col_s]
        g_rows = jnp.pad(g_rows, ((0, CH), (0, 0))).reshape(-1, 2 * D)
        le = _spmm(g_rows, dst_s, bounds, zro).reshape(-1, D)[:N_NODES]
        e, g = _dense(e, le, w1_w[i].T, w1_b[i].reshape(1, D),
                      w2_w[i].T, w2_b[i].reshape(1, D))
        gs.append(g)

    gidx = jnp.concatenate([
        u_id.astype(jnp.int32),
        (n_user + pos_item).astype(jnp.int32),
        (n_user + neg_item).astype(jnp.int32),
    ], axis=0)
    gidxp = (gidx >> 1).reshape(1, -1)
    gidxoff = (gidx & 1) * D
    gps = [g.reshape(N_NODES // 2, 2 * D) for g in gs]
    o0, o1, o2 = _gather_out(gps[0], gps[1], gps[2], gps[3], gidxp, gidxoff)
    return (o0, o1, o2)
